# Initial kernel scaffold; baseline (speedup 1.0000x reference)
import jax
import jax.numpy as jnp
from jax.experimental import pallas as pl


def kernel(edge_index, q_emb, entity_embs, relation_embs, topic_entity_one_hot, W1, b1, W2, b2):
    raise NotImplementedError("write your pallas kernel here")


# speedup vs baseline: 5.7486x; 5.7486x over previous
import functools

import jax
import jax.numpy as jnp
from jax import lax
from jax.experimental import pallas as pl
from jax.experimental.pallas import tpu as pltpu
from jax.experimental.pallas import tpu_sc as plsc

N = 10000
NP = 10240
E = 160000
EMB = 128
CHUNK = 128
ROWS = E // CHUNK
NC = 2
NS = 16
NPT = NP // NS
EBLK = 2000

DDE_BASE = ROWS // NS
DDE_EXTRA = ROWS - NS * DDE_BASE
GW_BASE = ROWS // (NC * NS)
GW_EXTRA = ROWS - NC * NS * GW_BASE


def _dde_body(ei_ref, topic4_ref, zeros16_ref, pe_ref,
              gidx_v, aidx_v, msg_v, node_v, acc_sh, sem):
    cid = lax.axis_index("c")
    sid = lax.axis_index("s")
    iota = lax.iota(jnp.int32, 16)
    mask01 = iota < 2
    constrow = jnp.where(iota == 2, 1.0, 0.0)
    mask2f = constrow

    start = sid * DDE_BASE + jnp.minimum(sid, DDE_EXTRA)
    cnt = DDE_BASE + jnp.where(sid < DDE_EXTRA, 1, 0)

    def chain(c):
        for r in range(2):
            pltpu.sync_copy(zeros16_ref.at[pl.ds(sid * NPT, NPT)],
                            acc_sh.at[pl.ds(sid * NPT, NPT)])
            plsc.subcore_barrier()

            table = topic4_ref if r == 0 else pe_ref.at[c, 0]

            def ebody(j, carry):
                off = (start + j) * CHUNK
                pltpu.sync_copy(ei_ref.at[c, pl.ds(off, CHUNK)], gidx_v)
                pltpu.sync_copy(ei_ref.at[1 - c, pl.ds(off, CHUNK)], aidx_v)
                pltpu.async_copy(table.at[gidx_v], msg_v, sem).wait()
                pltpu.sync_copy(msg_v, acc_sh.at[aidx_v], add=True)
                return carry
            lax.fori_loop(0, cnt, ebody, 0)
            plsc.subcore_barrier()

            pltpu.sync_copy(acc_sh.at[pl.ds(sid * NPT, NPT)], node_v)

            def dbody(i, carry):
                row = node_v[i, :]
                dvec = jnp.broadcast_to(row[2], (16,))
                inv = 1.0 / jnp.maximum(dvec, 1.0)
                node_v[i, :] = jnp.where(mask01, row * inv, constrow)
                return carry
            lax.fori_loop(0, NPT, dbody, 0)

            pltpu.sync_copy(node_v, pe_ref.at[c, r, pl.ds(sid * NPT, NPT)])
            plsc.subcore_barrier()

    @pl.when(cid == 0)
    def _():
        chain(0)

    @pl.when(cid == 1)
    def _():
        chain(1)


_dde = pl.kernel(
    _dde_body,
    out_type=jax.ShapeDtypeStruct((2, 2, NP, 16), jnp.float32),
    mesh=plsc.VectorSubcoreMesh(core_axis_name="c", subcore_axis_name="s"),
    scratch_types=[
        pltpu.VMEM((CHUNK,), jnp.int32),
        pltpu.VMEM((CHUNK,), jnp.int32),
        pltpu.VMEM((CHUNK, 16), jnp.float32),
        pltpu.VMEM((NPT, 16), jnp.float32),
        pltpu.VMEM_SHARED((NP, 16), jnp.float32),
        pltpu.SemaphoreType.DMA,
    ],
    compiler_params=pltpu.CompilerParams(use_tc_tiling_on_sc=False),
)


def _gather_body(ei_ref, zs_ref, zd_ref, g_ref,
                 idxs_v, idxd_v, bufa, bufb, sema, semb):
    cid = lax.axis_index("c")
    sid = lax.axis_index("s")
    wid = sid * NC + cid
    start = wid * GW_BASE + jnp.minimum(wid, GW_EXTRA)
    cnt = GW_BASE + jnp.where(wid < GW_EXTRA, 1, 0)

    def body(j, carry):
        off = (start + j) * CHUNK
        pltpu.sync_copy(ei_ref.at[0, pl.ds(off, CHUNK)], idxs_v)
        pltpu.sync_copy(ei_ref.at[1, pl.ds(off, CHUNK)], idxd_v)
        cpa = pltpu.async_copy(zs_ref.at[idxs_v], bufa, sema)
        cpb = pltpu.async_copy(zd_ref.at[idxd_v], bufb, semb)
        cpa.wait()
        cpb.wait()

        def add_row(i, c2):
            for k in range(EMB // 16):
                sl = pl.ds(k * 16, 16)
                bufa[i, sl] = bufa[i, sl] + bufb[i, sl]
            return c2
        lax.fori_loop(0, CHUNK, add_row, 0)
        pltpu.sync_copy(bufa, g_ref.at[pl.ds(off, CHUNK)])
        return carry
    lax.fori_loop(0, cnt, body, 0)


_gather = pl.kernel(
    _gather_body,
    out_type=jax.ShapeDtypeStruct((E, EMB), jnp.float32),
    mesh=plsc.VectorSubcoreMesh(core_axis_name="c", subcore_axis_name="s"),
    scratch_types=[
        pltpu.VMEM((CHUNK,), jnp.int32),
        pltpu.VMEM((CHUNK,), jnp.int32),
        pltpu.VMEM((CHUNK, EMB), jnp.float32),
        pltpu.VMEM((CHUNK, EMB), jnp.float32),
        pltpu.SemaphoreType.DMA,
        pltpu.SemaphoreType.DMA,
    ],
)


def _ztables_kernel(h_ref, q_ref, wq_ref, ws_ref, wd_ref, b1_ref,
                    zs_ref, zd_ref):
    qc = jnp.dot(q_ref[...], wq_ref[...],
                 preferred_element_type=jnp.float32) + b1_ref[...]
    h = h_ref[...]
    zs_ref[...] = jnp.dot(h, ws_ref[...],
                          preferred_element_type=jnp.float32) + qc
    zd_ref[...] = jnp.dot(h, wd_ref[...],
                          preferred_element_type=jnp.float32)


def _ztables(hpad, q_emb, Wq, Ws, Wd, b1row):
    return pl.pallas_call(
        _ztables_kernel,
        out_shape=(
            jax.ShapeDtypeStruct((N, EMB), jnp.float32),
            jax.ShapeDtypeStruct((N, EMB), jnp.float32),
        ),
    )(hpad, q_emb, Wq, Ws, Wd, b1row)


def _edge_mlp_kernel(rel_ref, g_ref, wr_ref, w2_ref, b2_ref, out_ref):
    z = jnp.dot(rel_ref[...], wr_ref[...],
                preferred_element_type=jnp.float32) + g_ref[...]
    h = jnp.maximum(z, 0.0)
    out_ref[...] = jnp.dot(h, w2_ref[...],
                           preferred_element_type=jnp.float32) + b2_ref[...]


def _edge_mlp(relation_embs, G, Wr, W2, b2row):
    nblk = E // EBLK
    return pl.pallas_call(
        _edge_mlp_kernel,
        grid=(nblk,),
        in_specs=[
            pl.BlockSpec((EBLK, EMB), lambda i: (i, 0)),
            pl.BlockSpec((EBLK, EMB), lambda i: (i, 0)),
            pl.BlockSpec((EMB, EMB), lambda i: (0, 0)),
            pl.BlockSpec((EMB, 1), lambda i: (0, 0)),
            pl.BlockSpec((1, 1), lambda i: (0, 0)),
        ],
        out_specs=pl.BlockSpec((EBLK, 1), lambda i: (i, 0)),
        out_shape=jax.ShapeDtypeStruct((E, 1), jnp.float32),
    )(relation_embs, G, Wr, W2, b2row)


def kernel(edge_index, q_emb, entity_embs, relation_embs,
           topic_entity_one_hot, W1, b1, W2, b2):
    ei = edge_index.astype(jnp.int32)
    topic4 = jnp.concatenate(
        [topic_entity_one_hot,
         jnp.ones((N, 1), jnp.float32),
         jnp.zeros((N, 13), jnp.float32)], axis=1)
    topic4 = jnp.pad(topic4, ((0, NP - N), (0, 0)))
    zeros16 = jnp.zeros((NP, 16), jnp.float32)

    pe = _dde(ei, topic4, zeros16)

    h_e = jnp.concatenate(
        [entity_embs, topic_entity_one_hot,
         pe[0, 0, :N, :2], pe[0, 1, :N, :2],
         pe[1, 0, :N, :2], pe[1, 1, :N, :2]], axis=1)
    hpad = jnp.pad(h_e, ((0, 0), (0, 6)))
    Ws = jnp.pad(W1[128:266], ((0, 6), (0, 0)))
    Wd = jnp.pad(W1[394:532], ((0, 6), (0, 0)))

    Zs, Zd = _ztables(hpad, q_emb, W1[0:128], Ws, Wd, b1.reshape(1, EMB))

    G = _gather(ei, Zs, Zd)

    return _edge_mlp(relation_embs, G, W1[266:394], W2, b2.reshape(1, 1))


# speedup vs baseline: 9.8195x; 1.7082x over previous
import functools

import jax
import jax.numpy as jnp
from jax import lax
from jax.experimental import pallas as pl
from jax.experimental.pallas import tpu as pltpu
from jax.experimental.pallas import tpu_sc as plsc

N = 10000
NP = 10240
E = 160000
EMB = 128
CHUNK = 128
ROWS = E // CHUNK
NC = 2
NS = 16
NPT = NP // NS
EBLK = 2000

DDE_BASE = ROWS // NS
DDE_EXTRA = ROWS - NS * DDE_BASE
GW_BASE = ROWS // (NC * NS)
GW_EXTRA = ROWS - NC * NS * GW_BASE

_SC_PARAMS = pltpu.CompilerParams(use_tc_tiling_on_sc=False)


def _reg_gather(x, idx):
    return lax.gather(
        x, idx[:, None],
        dimension_numbers=lax.GatherDimensionNumbers(
            offset_dims=(), collapsed_slice_dims=(0,), start_index_map=(0,)),
        slice_sizes=(1,),
        mode=lax.GatherScatterMode.PROMISE_IN_BOUNDS)


def _dde_body(ei_ref, topic4_ref, zeros16_ref, pe_ref,
              gidx_all, aidx_all, msg2, node_v, acc_sh, sem0, sem1):
    cid = lax.axis_index("c")
    sid = lax.axis_index("s")
    iota = lax.iota(jnp.int32, 16)
    mask01 = iota < 2
    constrow = jnp.where(iota == 2, 1.0, 0.0)

    start = sid * DDE_BASE + jnp.minimum(sid, DDE_EXTRA)
    cnt = DDE_BASE + jnp.where(sid < DDE_EXTRA, 1, 0)
    sems = (sem0, sem1)

    def chain(c):
        pltpu.sync_copy(ei_ref.at[c, pl.ds(start, DDE_BASE)],
                        gidx_all.at[pl.ds(0, DDE_BASE)])
        pltpu.sync_copy(ei_ref.at[1 - c, pl.ds(start, DDE_BASE)],
                        aidx_all.at[pl.ds(0, DDE_BASE)])

        @pl.when(sid < DDE_EXTRA)
        def _():
            pltpu.sync_copy(ei_ref.at[c, pl.ds(start + DDE_BASE, 1)],
                            gidx_all.at[pl.ds(DDE_BASE, 1)])
            pltpu.sync_copy(ei_ref.at[1 - c, pl.ds(start + DDE_BASE, 1)],
                            aidx_all.at[pl.ds(DDE_BASE, 1)])

        for r in range(2):
            pltpu.sync_copy(zeros16_ref.at[pl.ds(sid * NPT, NPT)],
                            acc_sh.at[pl.ds(sid * NPT, NPT)])
            plsc.subcore_barrier()

            table = topic4_ref if r == 0 else pe_ref.at[c, 0]

            def issue(j, s):
                pltpu.async_copy(table.at[gidx_all.at[j]], msg2.at[s],
                                 sems[s])

            def drain(s):
                pltpu.make_async_copy(table.at[pl.ds(0, CHUNK)],
                                      msg2.at[s], sems[s]).wait()

            def scatter(j, s):
                pltpu.sync_copy(msg2.at[s], acc_sh.at[aidx_all.at[j]],
                                add=True)

            issue(0, 0)
            issue(1, 1)

            def pair(j2, carry):
                a = 2 * j2
                drain(0)
                scatter(a, 0)

                @pl.when(a + 2 < cnt)
                def _():
                    issue(a + 2, 0)
                drain(1)
                scatter(a + 1, 1)

                @pl.when(a + 3 < cnt)
                def _():
                    issue(a + 3, 1)
                return carry
            lax.fori_loop(0, DDE_BASE // 2, pair, 0)

            @pl.when(sid < DDE_EXTRA)
            def _():
                drain(0)
                scatter(DDE_BASE, 0)

            plsc.subcore_barrier()

            pltpu.sync_copy(acc_sh.at[pl.ds(sid * NPT, NPT)], node_v)

            def dbody(i, carry):
                row = node_v[i, :]
                dvec = jnp.broadcast_to(row[2], (16,))
                inv = 1.0 / jnp.maximum(dvec, 1.0)
                node_v[i, :] = jnp.where(mask01, row * inv, constrow)
                return carry
            lax.fori_loop(0, NPT, dbody, 0)

            pltpu.sync_copy(node_v, pe_ref.at[c, r, pl.ds(sid * NPT, NPT)])
            plsc.subcore_barrier()

    @pl.when(cid == 0)
    def _():
        chain(0)

    @pl.when(cid == 1)
    def _():
        chain(1)


_dde = pl.kernel(
    _dde_body,
    out_type=jax.ShapeDtypeStruct((2, 2, NP, 16), jnp.float32),
    mesh=plsc.VectorSubcoreMesh(core_axis_name="c", subcore_axis_name="s"),
    scratch_types=[
        pltpu.VMEM((DDE_BASE + 1, CHUNK), jnp.int32),
        pltpu.VMEM((DDE_BASE + 1, CHUNK), jnp.int32),
        pltpu.VMEM((2, CHUNK, 16), jnp.float32),
        pltpu.VMEM((NPT, 16), jnp.float32),
        pltpu.VMEM_SHARED((NP, 16), jnp.float32),
        pltpu.SemaphoreType.DMA,
        pltpu.SemaphoreType.DMA,
    ],
    compiler_params=_SC_PARAMS,
)


def _gather_body(ei_ref, zs_ref, zd_ref, g_ref,
                 sidx_all, didx_all, abuf, bbuf, obuf,
                 sa0, sa1, sb0, sb1, sw0, sw1):
    cid = lax.axis_index("c")
    sid = lax.axis_index("s")
    wid = sid * NC + cid
    start = wid * GW_BASE + jnp.minimum(wid, GW_EXTRA)
    cnt = GW_BASE + jnp.where(wid < GW_EXTRA, 1, 0)
    sas = (sa0, sa1)
    sbs = (sb0, sb1)
    sws = (sw0, sw1)

    pltpu.sync_copy(ei_ref.at[0, pl.ds(start, GW_BASE)],
                    sidx_all.at[pl.ds(0, GW_BASE)])
    pltpu.sync_copy(ei_ref.at[1, pl.ds(start, GW_BASE)],
                    didx_all.at[pl.ds(0, GW_BASE)])

    @pl.when(wid < GW_EXTRA)
    def _():
        pltpu.sync_copy(ei_ref.at[0, pl.ds(start + GW_BASE, 1)],
                        sidx_all.at[pl.ds(GW_BASE, 1)])
        pltpu.sync_copy(ei_ref.at[1, pl.ds(start + GW_BASE, 1)],
                        didx_all.at[pl.ds(GW_BASE, 1)])

    def issue(j, s):
        pltpu.async_copy(zs_ref.at[sidx_all.at[j]], abuf.at[s], sas[s])
        pltpu.async_copy(zd_ref.at[didx_all.at[j]], bbuf.at[s], sbs[s])

    def drain_g(s):
        pltpu.make_async_copy(zs_ref.at[pl.ds(0, CHUNK)], abuf.at[s],
                              sas[s]).wait()
        pltpu.make_async_copy(zd_ref.at[pl.ds(0, CHUNK)], bbuf.at[s],
                              sbs[s]).wait()

    def drain_w(s):
        pltpu.make_async_copy(obuf.at[s], g_ref.at[pl.ds(0, CHUNK)],
                              sws[s]).wait()

    def process(j, s):
        drain_g(s)

        @pl.when(j >= 2)
        def _():
            drain_w(s)

        def add_row(i, c2):
            for k in range(EMB // 16):
                sl = pl.ds(k * 16, 16)
                obuf[s, i, sl] = abuf[s, i, sl] + bbuf[s, i, sl]
            return c2
        lax.fori_loop(0, CHUNK, add_row, 0)
        pltpu.async_copy(obuf.at[s], g_ref.at[pl.ds((start + j) * CHUNK,
                                                    CHUNK)], sws[s])

        @pl.when(j + 2 < cnt)
        def _():
            issue(j + 2, s)

    issue(0, 0)
    issue(1, 1)

    def pair(j2, carry):
        process(2 * j2, 0)
        process(2 * j2 + 1, 1)
        return carry
    lax.fori_loop(0, GW_BASE // 2, pair, 0)

    process(GW_BASE - 1, 0)

    @pl.when(wid < GW_EXTRA)
    def _():
        process(GW_BASE, 1)

    drain_w(0)
    drain_w(1)


_gather = pl.kernel(
    _gather_body,
    out_type=jax.ShapeDtypeStruct((E, EMB), jnp.float32),
    mesh=plsc.VectorSubcoreMesh(core_axis_name="c", subcore_axis_name="s"),
    scratch_types=[
        pltpu.VMEM((GW_BASE + 1, CHUNK), jnp.int32),
        pltpu.VMEM((GW_BASE + 1, CHUNK), jnp.int32),
        pltpu.VMEM((2, CHUNK, EMB), jnp.float32),
        pltpu.VMEM((2, CHUNK, EMB), jnp.float32),
        pltpu.VMEM((2, CHUNK, EMB), jnp.float32),
        pltpu.SemaphoreType.DMA,
        pltpu.SemaphoreType.DMA,
        pltpu.SemaphoreType.DMA,
        pltpu.SemaphoreType.DMA,
        pltpu.SemaphoreType.DMA,
        pltpu.SemaphoreType.DMA,
    ],
    compiler_params=_SC_PARAMS,
)


def _ztables_kernel(h_ref, q_ref, wq_ref, ws_ref, wd_ref, b1_ref,
                    zs_ref, zd_ref):
    qc = jnp.dot(q_ref[...], wq_ref[...],
                 preferred_element_type=jnp.float32) + b1_ref[...]
    h = h_ref[...]
    zs_ref[...] = jnp.dot(h, ws_ref[...],
                          preferred_element_type=jnp.float32) + qc
    zd_ref[...] = jnp.dot(h, wd_ref[...],
                          preferred_element_type=jnp.float32)


def _ztables(hpad, q_emb, Wq, Ws, Wd, b1row):
    return pl.pallas_call(
        _ztables_kernel,
        out_shape=(
            jax.ShapeDtypeStruct((N, EMB), jnp.float32),
            jax.ShapeDtypeStruct((N, EMB), jnp.float32),
        ),
    )(hpad, q_emb, Wq, Ws, Wd, b1row)


def _edge_mlp_kernel(rel_ref, g_ref, wr_ref, w2_ref, b2_ref, out_ref):
    z = jnp.dot(rel_ref[...], wr_ref[...],
                preferred_element_type=jnp.float32) + g_ref[...]
    h = jnp.maximum(z, 0.0)
    out_ref[...] = jnp.dot(h, w2_ref[...],
                           preferred_element_type=jnp.float32) + b2_ref[...]


def _edge_mlp(relation_embs, G, Wr, W2, b2row):
    nblk = E // EBLK
    return pl.pallas_call(
        _edge_mlp_kernel,
        grid=(nblk,),
        in_specs=[
            pl.BlockSpec((EBLK, EMB), lambda i: (i, 0)),
            pl.BlockSpec((EBLK, EMB), lambda i: (i, 0)),
            pl.BlockSpec((EMB, EMB), lambda i: (0, 0)),
            pl.BlockSpec((EMB, 1), lambda i: (0, 0)),
            pl.BlockSpec((1, 1), lambda i: (0, 0)),
        ],
        out_specs=pl.BlockSpec((EBLK, 1), lambda i: (i, 0)),
        out_shape=jax.ShapeDtypeStruct((E, 1), jnp.float32),
    )(relation_embs, G, Wr, W2, b2row)


def kernel(edge_index, q_emb, entity_embs, relation_embs,
           topic_entity_one_hot, W1, b1, W2, b2):
    ei = edge_index.astype(jnp.int32).reshape(2, ROWS, CHUNK)
    topic4 = jnp.concatenate(
        [topic_entity_one_hot,
         jnp.ones((N, 1), jnp.float32),
         jnp.zeros((N, 13), jnp.float32)], axis=1)
    topic4 = jnp.pad(topic4, ((0, NP - N), (0, 0)))
    zeros16 = jnp.zeros((NP, 16), jnp.float32)

    pe = _dde(ei, topic4, zeros16)

    h_e = jnp.concatenate(
        [entity_embs, topic_entity_one_hot,
         pe[0, 0, :N, :2], pe[0, 1, :N, :2],
         pe[1, 0, :N, :2], pe[1, 1, :N, :2]], axis=1)
    hpad = jnp.pad(h_e, ((0, 0), (0, 6)))
    Ws = jnp.pad(W1[128:266], ((0, 6), (0, 0)))
    Wd = jnp.pad(W1[394:532], ((0, 6), (0, 0)))

    Zs, Zd = _ztables(hpad, q_emb, W1[0:128], Ws, Wd, b1.reshape(1, EMB))

    G = _gather(ei, Zs, Zd)

    return _edge_mlp(relation_embs, G, W1[266:394], W2, b2.reshape(1, 1))
